# baseline (speedup 1.0000x reference)
import jax
import jax.numpy as jnp
from jax import lax
from jax.experimental import pallas as pl
from jax.experimental.pallas import tpu as pltpu
from jax.experimental.pallas import tpu_sc as plsc

N = 10000
E = 320000
D_IN = 128
H1, C1 = 8, 8
C2 = 64

NW = 32
B = 96
EPW = 10080
E_PAD = NW * EPW
NCH = EPW // B
N_ACC = 10112
ROWS_PER_SUB = N_ACC // 16
ACC_W = 80

_f32 = jnp.float32


def _dense1_body(x_ref, w_ref, asm_ref, adm_ref, pcm_ref, hs_ref, sa_ref):
    h = jnp.dot(x_ref[...], w_ref[...], preferred_element_type=_f32)
    hs_ref[...] = jnp.dot(h, pcm_ref[...], preferred_element_type=_f32)
    a_s = jnp.dot(h, asm_ref[...], preferred_element_type=_f32)
    a_d = jnp.dot(h, adm_ref[...], preferred_element_type=_f32)
    sa_ref[...] = jnp.concatenate([a_s, a_d], axis=1)


def _combine1_body(p_ref, sa_ref, hs_ref, w2_ref, a2_ref, b1_ref, r8_ref,
                   pt_ref, hs2_ref, sa2_ref):
    p = p_ref[0] + p_ref[1]
    pt = pt_ref[...]
    num = jnp.dot(p[:, :64], pt, preferred_element_type=_f32)
    den8 = p[:, 64:72]
    sa = sa_ref[...]
    t = sa[:, :8] + sa[:, 8:]
    exs = jnp.exp(jnp.where(t > 0, t, 0.2 * t))
    den = den8 + exs
    h1 = jnp.dot(hs_ref[...], pt, preferred_element_type=_f32)
    r8 = r8_ref[...]
    num = num + h1 * jnp.dot(exs, r8, preferred_element_type=_f32)
    den64 = jnp.dot(den, r8, preferred_element_type=_f32) + 1e-16
    out1 = num / den64 + b1_ref[...]
    x2 = jnp.where(out1 > 0, out1, jnp.exp(out1) - 1.0)
    h2 = jnp.dot(x2, w2_ref[...], preferred_element_type=_f32)
    hs2_ref[...] = h2
    sa2_ref[...] = jnp.dot(h2, a2_ref[...], preferred_element_type=_f32)


def _combine2_body(p_ref, sa2_ref, hs2_ref, b2_ref, out_ref):
    p = p_ref[0] + p_ref[1]
    num = p[:, :64]
    den = p[:, 64:65]
    sa = sa2_ref[...]
    t = sa[:, :1] + sa[:, 8:9]
    ex = jnp.exp(jnp.where(t > 0, t, 0.2 * t))
    den = den + ex + 1e-16
    out_ref[...] = (num + hs2_ref[...] * ex) / den + b2_ref[...]


_DNUMS = lax.GatherDimensionNumbers(
    offset_dims=(), collapsed_slice_dims=(0,), start_index_map=(0,))


def _shuf(v, idx):
    return lax.gather(v, idx[:, None], _DNUMS, (1,),
                      mode=lax.GatherScatterMode.PROMISE_IN_BOUNDS)


def _sc_edges(src_hbm, dst_hbm, sa_hbm, hs_hbm, z_hbm, out_hbm,
              sidx0, didx0, sas0, sad0, hb0,
              sidx1, didx1, sas1, sad1, hb1,
              stage0, stage1, sdix0, sdix1, acc,
              isem0, isem1, gs0, gd0, gh0, gs1, gd1, gh1,
              ssem0, ssem1):
    c = lax.axis_index("c")
    s = lax.axis_index("s")
    w = c * 16 + s
    lanes = lax.iota(jnp.int32, 16)

    row0 = s * ROWS_PER_SUB
    pltpu.sync_copy(z_hbm.at[pl.ds(row0, ROWS_PER_SUB)],
                    acc.at[pl.ds(row0, ROWS_PER_SUB)])
    plsc.subcore_barrier()

    base = w * EPW
    swap8 = lanes ^ 8
    low8 = lanes & 7

    bufs = (
        (sidx0, didx0, sas0, sad0, hb0, isem0, gs0, gd0, gh0),
        (sidx1, didx1, sas1, sad1, hb1, isem1, gs1, gd1, gh1),
    )
    stages = (stage0, stage1)
    sdixs = (sdix0, sdix1)
    ssems = (ssem0, ssem1)

    def issue_idx(p, off):
        si, di = bufs[p][0], bufs[p][1]
        isem = bufs[p][5]
        pltpu.async_copy(src_hbm.at[pl.ds(off, B)], si, isem)
        pltpu.async_copy(dst_hbm.at[pl.ds(off, B)], di, isem)

    def wait_idx(p):
        si, di = bufs[p][0], bufs[p][1]
        isem = bufs[p][5]
        pltpu.make_async_copy(src_hbm.at[pl.ds(0, B)], si, isem).wait()
        pltpu.make_async_copy(dst_hbm.at[pl.ds(0, B)], di, isem).wait()

    def issue_gathers(p):
        si, di, sas, sad, hb, _, gs, gd, gh = bufs[p]
        pltpu.async_copy(sa_hbm.at[si], sas, gs)
        pltpu.async_copy(sa_hbm.at[di], sad, gd)
        pltpu.async_copy(hs_hbm.at[si], hb, gh)

    def wait_gathers(p):
        si, di, sas, sad, hb, _, gs, gd, gh = bufs[p]
        pltpu.make_async_copy(sa_hbm.at[si], sas, gs).wait()
        pltpu.make_async_copy(sa_hbm.at[di], sad, gd).wait()
        pltpu.make_async_copy(hs_hbm.at[si], hb, gh).wait()

    def compute(p):
        sas, sad, hb = bufs[p][2], bufs[p][3], bufs[p][4]
        stage = stages[p]
        for e in range(B):
            vs = sas[e, :]
            vd = sad[e, :]
            t = vs + _shuf(vd, swap8)
            ex = jnp.exp(jnp.maximum(t, 0.2 * t))
            stage[e, pl.ds(64, 16)] = ex
        for e in range(B):
            w_v = _shuf(stage[e, pl.ds(64, 16)], low8)
            for kk in range(4):
                stage[e, pl.ds(kk * 16, 16)] = (
                    hb[e, pl.ds(kk * 16, 16)] * w_v)

    def wait_scatter(p):
        pltpu.make_async_copy(stages[p], acc.at[sdixs[p]], ssems[p]).wait()

    def issue_scatter(p):
        didx = bufs[p][1]
        sdix = sdixs[p]
        for g in range(B // 16):
            sdix[pl.ds(g * 16, 16)] = didx[pl.ds(g * 16, 16)]
        pltpu.async_copy(stages[p], acc.at[sdix], ssems[p], add=True)

    def step(k, p):
        q = 1 - p
        wait_idx(q)
        issue_gathers(q)
        wait_gathers(p)

        @pl.when(k >= 2)
        def _():
            wait_scatter(p)

        compute(p)
        issue_scatter(p)
        off2 = base + jnp.minimum(k + 2, NCH - 1) * B
        issue_idx(p, off2)

    issue_idx(0, base)
    wait_idx(0)
    issue_gathers(0)
    issue_idx(1, base + B)

    def pair(i, carry):
        step(2 * i, 0)
        step(2 * i + 1, 1)
        return carry

    lax.fori_loop(0, (NCH - 1) // 2, pair, 0)

    wait_idx(1)
    wait_gathers(0)
    wait_scatter(0)
    compute(0)
    issue_scatter(0)
    wait_scatter(1)
    wait_scatter(0)
    plsc.subcore_barrier()
    pltpu.sync_copy(acc.at[pl.ds(row0, ROWS_PER_SUB)],
                    out_hbm.at[c, pl.ds(row0, ROWS_PER_SUB)])


_BLK = 400
_GRID = N // _BLK


def _full(shape):
    return pl.BlockSpec(shape, lambda i: tuple(0 for _ in shape))


def _sc_call(srcp, dstp, sap, hsp, zacc):
    mesh = plsc.VectorSubcoreMesh(core_axis_name="c", subcore_axis_name="s")
    return pl.kernel(
        _sc_edges,
        out_type=jax.ShapeDtypeStruct((2, N_ACC, ACC_W), _f32),
        mesh=mesh,
        compiler_params=pltpu.CompilerParams(use_tc_tiling_on_sc=False),
        scratch_types=(
            [pltpu.VMEM((B,), jnp.int32), pltpu.VMEM((B,), jnp.int32),
             pltpu.VMEM((B, 16), _f32), pltpu.VMEM((B, 16), _f32),
             pltpu.VMEM((B, 64), _f32)] * 2
            + [pltpu.VMEM((B, ACC_W), _f32), pltpu.VMEM((B, ACC_W), _f32),
               pltpu.VMEM((B,), jnp.int32), pltpu.VMEM((B,), jnp.int32),
               pltpu.VMEM_SHARED((N_ACC, ACC_W), _f32)]
            + [pltpu.SemaphoreType.DMA] * 10
        ),
    )(srcp, dstp, sap, hsp, zacc)


def kernel(x, edge_index, W1, a_src1, a_dst1, b1, W2, a_src2, a_dst2, b2):
    src = edge_index[0]
    dst = edge_index[1]
    pad = E_PAD - E
    srcp = jnp.concatenate([src, jnp.zeros((pad,), jnp.int32)])
    dstp = jnp.concatenate([dst, jnp.full((pad,), N, jnp.int32)])
    eye8 = jnp.eye(8, dtype=_f32)
    asm1 = (a_src1[:, :, None] * eye8[:, None, :]).reshape(64, 8)
    adm1 = (a_dst1[:, :, None] * eye8[:, None, :]).reshape(64, 8)
    r8 = jnp.repeat(eye8, 8, axis=1)
    a2cat = jnp.concatenate(
        [jnp.tile(a_src2.reshape(64, 1), (1, 8)),
         jnp.tile(a_dst2.reshape(64, 1), (1, 8))], axis=1)
    b1row = b1.reshape(1, 64)
    b2row = b2.reshape(1, 64)
    ii = jnp.arange(64)
    perm = jnp.zeros((64, 64), _f32).at[ii, (ii % 8) * 8 + ii // 8].set(1.0)
    zacc = jnp.zeros((N_ACC, ACC_W), _f32)

    hs1, sa1 = pl.pallas_call(
        _dense1_body,
        grid=(_GRID,),
        in_specs=[
            pl.BlockSpec((_BLK, D_IN), lambda i: (i, 0)),
            _full((D_IN, 64)), _full((64, 8)), _full((64, 8)),
            _full((64, 64)),
        ],
        out_specs=[
            pl.BlockSpec((_BLK, 64), lambda i: (i, 0)),
            pl.BlockSpec((_BLK, 16), lambda i: (i, 0)),
        ],
        out_shape=[
            jax.ShapeDtypeStruct((N, 64), _f32),
            jax.ShapeDtypeStruct((N, 16), _f32),
        ],
    )(x, W1, asm1, adm1, perm)

    hs1p = jnp.pad(hs1, ((0, N_ACC - N), (0, 0)))
    sa1p = jnp.pad(sa1, ((0, N_ACC - N), (0, 0)))

    p1 = _sc_call(srcp, dstp, sa1p, hs1p, zacc)

    hs2, sa2 = pl.pallas_call(
        _combine1_body,
        grid=(_GRID,),
        in_specs=[
            pl.BlockSpec((2, _BLK, ACC_W), lambda i: (0, i, 0)),
            pl.BlockSpec((_BLK, 16), lambda i: (i, 0)),
            pl.BlockSpec((_BLK, 64), lambda i: (i, 0)),
            _full((64, 64)), _full((64, 16)), _full((1, 64)), _full((8, 64)),
            _full((64, 64)),
        ],
        out_specs=[
            pl.BlockSpec((_BLK, 64), lambda i: (i, 0)),
            pl.BlockSpec((_BLK, 16), lambda i: (i, 0)),
        ],
        out_shape=[
            jax.ShapeDtypeStruct((N, 64), _f32),
            jax.ShapeDtypeStruct((N, 16), _f32),
        ],
    )(p1, sa1, hs1, W2, a2cat, b1row, r8, perm.T)

    hs2p = jnp.pad(hs2, ((0, N_ACC - N), (0, 0)))
    sa2p = jnp.pad(sa2, ((0, N_ACC - N), (0, 0)))

    p2 = _sc_call(srcp, dstp, sa2p, hs2p, zacc)

    out = pl.pallas_call(
        _combine2_body,
        grid=(_GRID,),
        in_specs=[
            pl.BlockSpec((2, _BLK, ACC_W), lambda i: (0, i, 0)),
            pl.BlockSpec((_BLK, 16), lambda i: (i, 0)),
            pl.BlockSpec((_BLK, 64), lambda i: (i, 0)),
            _full((1, 64)),
        ],
        out_specs=pl.BlockSpec((_BLK, 64), lambda i: (i, 0)),
        out_shape=jax.ShapeDtypeStruct((N, 64), _f32),
    )(p2, sa2, hs2, b2row)

    return out


# speedup vs baseline: 1.0911x; 1.0911x over previous
import jax
import jax.numpy as jnp
from jax import lax
from jax.experimental import pallas as pl
from jax.experimental.pallas import tpu as pltpu
from jax.experimental.pallas import tpu_sc as plsc

N = 10000
E = 320000
D_IN = 128
H1, C1 = 8, 8
C2 = 64

NW = 32
B = 96
EPW = 10080
E_PAD = NW * EPW
NCH = EPW // B
N_ACC = 10112
ROWS_PER_SUB = N_ACC // 16
ACC_W = 80

_f32 = jnp.float32


def _dense1_body(x_ref, w_ref, asm_ref, adm_ref, pcm_ref, hs_ref, sa_ref):
    h = jnp.dot(x_ref[...], w_ref[...], preferred_element_type=_f32)
    hs_ref[...] = jnp.dot(h, pcm_ref[...], preferred_element_type=_f32)
    a_s = jnp.dot(h, asm_ref[...], preferred_element_type=_f32)
    a_d = jnp.dot(h, adm_ref[...], preferred_element_type=_f32)
    sa_ref[...] = jnp.concatenate([a_s, a_d], axis=1)


def _combine1_body(p_ref, sa_ref, hs_ref, w2_ref, a2_ref, b1_ref, r8_ref,
                   pt_ref, hs2_ref, sa2_ref):
    p = p_ref[0] + p_ref[1]
    pt = pt_ref[...]
    num = jnp.dot(p[:, :64], pt, preferred_element_type=_f32)
    den8 = p[:, 64:72]
    sa = sa_ref[...]
    t = sa[:, :8] + sa[:, 8:]
    exs = jnp.exp(jnp.where(t > 0, t, 0.2 * t))
    den = den8 + exs
    h1 = jnp.dot(hs_ref[...], pt, preferred_element_type=_f32)
    r8 = r8_ref[...]
    num = num + h1 * jnp.dot(exs, r8, preferred_element_type=_f32)
    den64 = jnp.dot(den, r8, preferred_element_type=_f32) + 1e-16
    out1 = num / den64 + b1_ref[...]
    x2 = jnp.where(out1 > 0, out1, jnp.exp(out1) - 1.0)
    h2 = jnp.dot(x2, w2_ref[...], preferred_element_type=_f32)
    hs2_ref[...] = h2
    sa2_ref[...] = jnp.dot(h2, a2_ref[...], preferred_element_type=_f32)


def _combine2_body(p_ref, sa2_ref, hs2_ref, b2_ref, out_ref):
    p = p_ref[0] + p_ref[1]
    num = p[:, :64]
    den = p[:, 64:65]
    sa = sa2_ref[...]
    t = sa[:, :1] + sa[:, 8:9]
    ex = jnp.exp(jnp.where(t > 0, t, 0.2 * t))
    den = den + ex + 1e-16
    out_ref[...] = (num + hs2_ref[...] * ex) / den + b2_ref[...]


_DNUMS = lax.GatherDimensionNumbers(
    offset_dims=(), collapsed_slice_dims=(0,), start_index_map=(0,))


def _shuf(v, idx):
    return lax.gather(v, idx[:, None], _DNUMS, (1,),
                      mode=lax.GatherScatterMode.PROMISE_IN_BOUNDS)


def _sc_edges(src_hbm, dst_hbm, sa_hbm, hs_hbm, z_hbm, out_hbm,
              sidx0, didx0, sas0, sad0, hb0,
              sidx1, didx1, sas1, sad1, hb1,
              stage0, stage1, sdix0, sdix1, acc,
              isem0, isem1, gs0, gd0, gh0, gs1, gd1, gh1,
              ssem0, ssem1):
    c = lax.axis_index("c")
    s = lax.axis_index("s")
    w = c * 16 + s
    lanes = lax.iota(jnp.int32, 16)

    row0 = s * ROWS_PER_SUB
    pltpu.sync_copy(z_hbm.at[pl.ds(row0, ROWS_PER_SUB)],
                    acc.at[pl.ds(row0, ROWS_PER_SUB)])
    plsc.subcore_barrier()

    base = w * EPW
    swap8 = lanes ^ 8
    low8 = lanes & 7

    bufs = (
        (sidx0, didx0, sas0, sad0, hb0, isem0, gs0, gd0, gh0),
        (sidx1, didx1, sas1, sad1, hb1, isem1, gs1, gd1, gh1),
    )
    stages = (stage0, stage1)
    sdixs = (sdix0, sdix1)
    ssems = (ssem0, ssem1)

    def issue_idx(p, off):
        si, di = bufs[p][0], bufs[p][1]
        isem = bufs[p][5]
        pltpu.async_copy(src_hbm.at[pl.ds(off, B)], si, isem)
        pltpu.async_copy(dst_hbm.at[pl.ds(off, B)], di, isem)

    def wait_idx(p):
        si, di = bufs[p][0], bufs[p][1]
        isem = bufs[p][5]
        pltpu.make_async_copy(src_hbm.at[pl.ds(0, B)], si, isem).wait()
        pltpu.make_async_copy(dst_hbm.at[pl.ds(0, B)], di, isem).wait()

    def issue_gathers(p):
        si, di, sas, sad, hb, _, gs, gd, gh = bufs[p]
        pltpu.async_copy(sa_hbm.at[si], sas, gs)
        pltpu.async_copy(sa_hbm.at[di], sad, gd)
        pltpu.async_copy(hs_hbm.at[si], hb, gh)

    def wait_gathers(p):
        si, di, sas, sad, hb, _, gs, gd, gh = bufs[p]
        pltpu.make_async_copy(sa_hbm.at[si], sas, gs).wait()
        pltpu.make_async_copy(sa_hbm.at[di], sad, gd).wait()
        pltpu.make_async_copy(hs_hbm.at[si], hb, gh).wait()

    def compute(p):
        sas, sad, hb = bufs[p][2], bufs[p][3], bufs[p][4]
        stage = stages[p]
        for e in range(B):
            vs = sas[e, :]
            vd = sad[e, :]
            t = vs + _shuf(vd, swap8)
            ex = jnp.exp(jnp.maximum(t, 0.2 * t))
            stage[e, pl.ds(64, 16)] = ex
            w_v = _shuf(ex, low8)
            for kk in range(4):
                stage[e, pl.ds(kk * 16, 16)] = (
                    hb[e, pl.ds(kk * 16, 16)] * w_v)

    def wait_scatter(p):
        pltpu.make_async_copy(stages[p], acc.at[sdixs[p]], ssems[p]).wait()

    def issue_scatter(p):
        didx = bufs[p][1]
        sdix = sdixs[p]
        for g in range(B // 16):
            sdix[pl.ds(g * 16, 16)] = didx[pl.ds(g * 16, 16)]
        pltpu.async_copy(stages[p], acc.at[sdix], ssems[p], add=True)

    def step(k, p):
        q = 1 - p
        wait_idx(q)
        issue_gathers(q)
        wait_gathers(p)

        @pl.when(k >= 2)
        def _():
            wait_scatter(p)

        compute(p)
        issue_scatter(p)
        off2 = base + jnp.minimum(k + 2, NCH - 1) * B
        issue_idx(p, off2)

    issue_idx(0, base)
    wait_idx(0)
    issue_gathers(0)
    issue_idx(1, base + B)

    def pair(i, carry):
        step(2 * i, 0)
        step(2 * i + 1, 1)
        return carry

    lax.fori_loop(0, (NCH - 1) // 2, pair, 0)

    wait_idx(1)
    wait_gathers(0)
    wait_scatter(0)
    compute(0)
    issue_scatter(0)
    wait_scatter(1)
    wait_scatter(0)
    plsc.subcore_barrier()
    pltpu.sync_copy(acc.at[pl.ds(row0, ROWS_PER_SUB)],
                    out_hbm.at[c, pl.ds(row0, ROWS_PER_SUB)])


_BLK = 400
_GRID = N // _BLK


def _full(shape):
    return pl.BlockSpec(shape, lambda i: tuple(0 for _ in shape))


def _sc_call(srcp, dstp, sap, hsp, zacc):
    mesh = plsc.VectorSubcoreMesh(core_axis_name="c", subcore_axis_name="s")
    return pl.kernel(
        _sc_edges,
        out_type=jax.ShapeDtypeStruct((2, N_ACC, ACC_W), _f32),
        mesh=mesh,
        compiler_params=pltpu.CompilerParams(use_tc_tiling_on_sc=False),
        scratch_types=(
            [pltpu.VMEM((B,), jnp.int32), pltpu.VMEM((B,), jnp.int32),
             pltpu.VMEM((B, 16), _f32), pltpu.VMEM((B, 16), _f32),
             pltpu.VMEM((B, 64), _f32)] * 2
            + [pltpu.VMEM((B, ACC_W), _f32), pltpu.VMEM((B, ACC_W), _f32),
               pltpu.VMEM((B,), jnp.int32), pltpu.VMEM((B,), jnp.int32),
               pltpu.VMEM_SHARED((N_ACC, ACC_W), _f32)]
            + [pltpu.SemaphoreType.DMA] * 10
        ),
    )(srcp, dstp, sap, hsp, zacc)


def kernel(x, edge_index, W1, a_src1, a_dst1, b1, W2, a_src2, a_dst2, b2):
    src = edge_index[0]
    dst = edge_index[1]
    pad = E_PAD - E
    srcp = jnp.concatenate([src, jnp.zeros((pad,), jnp.int32)])
    dstp = jnp.concatenate([dst, jnp.full((pad,), N, jnp.int32)])
    eye8 = jnp.eye(8, dtype=_f32)
    asm1 = (a_src1[:, :, None] * eye8[:, None, :]).reshape(64, 8)
    adm1 = (a_dst1[:, :, None] * eye8[:, None, :]).reshape(64, 8)
    r8 = jnp.repeat(eye8, 8, axis=1)
    a2cat = jnp.concatenate(
        [jnp.tile(a_src2.reshape(64, 1), (1, 8)),
         jnp.tile(a_dst2.reshape(64, 1), (1, 8))], axis=1)
    b1row = b1.reshape(1, 64)
    b2row = b2.reshape(1, 64)
    ii = jnp.arange(64)
    perm = jnp.zeros((64, 64), _f32).at[ii, (ii % 8) * 8 + ii // 8].set(1.0)
    zacc = jnp.zeros((N_ACC, ACC_W), _f32)

    hs1, sa1 = pl.pallas_call(
        _dense1_body,
        grid=(_GRID,),
        in_specs=[
            pl.BlockSpec((_BLK, D_IN), lambda i: (i, 0)),
            _full((D_IN, 64)), _full((64, 8)), _full((64, 8)),
            _full((64, 64)),
        ],
        out_specs=[
            pl.BlockSpec((_BLK, 64), lambda i: (i, 0)),
            pl.BlockSpec((_BLK, 16), lambda i: (i, 0)),
        ],
        out_shape=[
            jax.ShapeDtypeStruct((N, 64), _f32),
            jax.ShapeDtypeStruct((N, 16), _f32),
        ],
    )(x, W1, asm1, adm1, perm)

    hs1p = jnp.pad(hs1, ((0, N_ACC - N), (0, 0)))
    sa1p = jnp.pad(sa1, ((0, N_ACC - N), (0, 0)))

    p1 = _sc_call(srcp, dstp, sa1p, hs1p, zacc)

    hs2, sa2 = pl.pallas_call(
        _combine1_body,
        grid=(_GRID,),
        in_specs=[
            pl.BlockSpec((2, _BLK, ACC_W), lambda i: (0, i, 0)),
            pl.BlockSpec((_BLK, 16), lambda i: (i, 0)),
            pl.BlockSpec((_BLK, 64), lambda i: (i, 0)),
            _full((64, 64)), _full((64, 16)), _full((1, 64)), _full((8, 64)),
            _full((64, 64)),
        ],
        out_specs=[
            pl.BlockSpec((_BLK, 64), lambda i: (i, 0)),
            pl.BlockSpec((_BLK, 16), lambda i: (i, 0)),
        ],
        out_shape=[
            jax.ShapeDtypeStruct((N, 64), _f32),
            jax.ShapeDtypeStruct((N, 16), _f32),
        ],
    )(p1, sa1, hs1, W2, a2cat, b1row, r8, perm.T)

    hs2p = jnp.pad(hs2, ((0, N_ACC - N), (0, 0)))
    sa2p = jnp.pad(sa2, ((0, N_ACC - N), (0, 0)))

    p2 = _sc_call(srcp, dstp, sa2p, hs2p, zacc)

    out = pl.pallas_call(
        _combine2_body,
        grid=(_GRID,),
        in_specs=[
            pl.BlockSpec((2, _BLK, ACC_W), lambda i: (0, i, 0)),
            pl.BlockSpec((_BLK, 16), lambda i: (i, 0)),
            pl.BlockSpec((_BLK, 64), lambda i: (i, 0)),
            _full((1, 64)),
        ],
        out_specs=pl.BlockSpec((_BLK, 64), lambda i: (i, 0)),
        out_shape=jax.ShapeDtypeStruct((N, 64), _f32),
    )(p2, sa2, hs2, b2row)

    return out
